# baseline (speedup 1.0000x reference)
import jax
import jax.numpy as jnp
from jax import lax
from jax.experimental import pallas as pl
from jax.experimental.pallas import tpu as pltpu
from jax.experimental.pallas import tpu_sc as plsc

_D = 32
_W = 128
_K = 8
_L = 16


def _build(batch, seq):
    n = batch * seq
    num_rows = n // _W
    mesh = plsc.VectorSubcoreMesh(core_axis_name="c", subcore_axis_name="s")

    @jax.jit
    def run(table, input_text):
        idx2d = input_text.T.reshape(num_rows, _W)

        @pl.kernel(
            out_type=jax.ShapeDtypeStruct((n, _D), jnp.float32),
            mesh=mesh,
            scratch_types=[pltpu.SemaphoreType.DMA],
            compiler_params=pltpu.CompilerParams(use_tc_tiling_on_sc=False),
        )
        def gather_k(table_hbm, i_hbm, y_hbm, sem):
            def body(i_vmem, y_vmem):
                copies = [
                    pltpu.async_copy(
                        table_hbm.at[i_vmem.at[j]],
                        y_vmem.at[pl.ds(j * _W, _W)],
                        sem,
                    )
                    for j in range(_K)
                ]
                for c in copies:
                    c.wait()

            pltpu.emit_pipeline(
                body,
                grid=(num_rows // _K,),
                in_specs=[pl.BlockSpec((_K, _W), lambda i: (i, 0))],
                out_specs=[pl.BlockSpec((_K * _W, _D), lambda i: (i, 0))],
                core_axis_name=("c", "s"),
                dimension_semantics=(pltpu.PARALLEL,),
            )(i_hbm, y_hbm)

        y = gather_k(table, idx2d).reshape(n * _D)

        @pl.kernel(
            out_type=jax.ShapeDtypeStruct((seq, _D, batch), jnp.float32),
            mesh=mesh,
            compiler_params=pltpu.CompilerParams(
                use_tc_tiling_on_sc=True, needs_layout_passes=False
            ),
        )
        def transpose_k(y_hbm, x_hbm):
            def body(y_vmem, x_vmem):
                base = lax.iota(jnp.int32, _L) * _D
                for d in range(_D):
                    for j in range(_W // _L):
                        vals = plsc.load_gather(
                            y_vmem, [base + (j * (_L * _D) + d)]
                        )
                        x_vmem[0, d, pl.ds(j * _L, _L)] = vals

            pltpu.emit_pipeline(
                body,
                grid=(num_rows,),
                in_specs=[pl.BlockSpec((_W * _D,), lambda i: (i,))],
                out_specs=[
                    pl.BlockSpec(
                        (1, _D, _W),
                        lambda i: (i // (batch // _W), 0, i % (batch // _W)),
                    )
                ],
                core_axis_name=("c", "s"),
                dimension_semantics=(pltpu.PARALLEL,),
            )(y_hbm, x_hbm)

        x = transpose_k(y)
        return jnp.transpose(x, (2, 0, 1))

    return run


def kernel(input_text, embedding_table):
    batch, seq = input_text.shape
    run = _build(batch, seq)
    return run(embedding_table, input_text)


# speedup vs baseline: 1.2757x; 1.2757x over previous
import jax
import jax.numpy as jnp
from jax import lax
from jax.experimental import pallas as pl
from jax.experimental.pallas import tpu as pltpu
from jax.experimental.pallas import tpu_sc as plsc

_D = 32
_W = 128
_K = 8
_L = 16


def _build(batch, seq):
    n = batch * seq
    num_rows = n // _W
    mesh = plsc.VectorSubcoreMesh(core_axis_name="c", subcore_axis_name="s")

    @jax.jit
    def run(table, input_text):
        idx2d = input_text.T.reshape(num_rows, _W)

        @pl.kernel(
            out_type=jax.ShapeDtypeStruct((n, _D), jnp.float32),
            mesh=mesh,
            scratch_types=[pltpu.SemaphoreType.DMA],
            compiler_params=pltpu.CompilerParams(use_tc_tiling_on_sc=False),
        )
        def gather_k(table_hbm, i_hbm, y_hbm, sem):
            def body(i_vmem, y_vmem):
                copies = [
                    pltpu.async_copy(
                        table_hbm.at[i_vmem.at[j]],
                        y_vmem.at[pl.ds(j * _W, _W)],
                        sem,
                    )
                    for j in range(_K)
                ]
                for c in copies:
                    c.wait()

            pltpu.emit_pipeline(
                body,
                grid=(num_rows // _K,),
                in_specs=[pl.BlockSpec((_K, _W), lambda i: (i, 0))],
                out_specs=[pl.BlockSpec((_K * _W, _D), lambda i: (i, 0))],
                core_axis_name=("c", "s"),
                dimension_semantics=(pltpu.PARALLEL,),
            )(i_hbm, y_hbm)

        y = gather_k(table, idx2d).reshape(n * _D)

        @pl.kernel(
            out_type=jax.ShapeDtypeStruct((seq, _D, batch), jnp.float32),
            mesh=mesh,
            compiler_params=pltpu.CompilerParams(
                use_tc_tiling_on_sc=True, needs_layout_passes=False
            ),
        )
        def transpose_k(y_hbm, x_hbm):
            def body(y_vmem, x_vmem):
                base = lax.iota(jnp.int32, _L) * _D
                nj = _W // _L

                @plsc.parallel_loop(0, _D * nj, unroll=8)
                def _(i):
                    d = i // nj
                    j = i % nj
                    vals = plsc.load_gather(
                        y_vmem, [base + (j * (_L * _D) + d)]
                    )
                    x_vmem[0, d, pl.ds(j * _L, _L)] = vals

            pltpu.emit_pipeline(
                body,
                grid=(num_rows,),
                in_specs=[pl.BlockSpec((_W * _D,), lambda i: (i,))],
                out_specs=[
                    pl.BlockSpec(
                        (1, _D, _W),
                        lambda i: (i // (batch // _W), 0, i % (batch // _W)),
                    )
                ],
                core_axis_name=("c", "s"),
                dimension_semantics=(pltpu.PARALLEL,),
            )(y_hbm, x_hbm)

        x = transpose_k(y)
        return jnp.transpose(x, (2, 0, 1))

    return run


def kernel(input_text, embedding_table):
    batch, seq = input_text.shape
    run = _build(batch, seq)
    return run(embedding_table, input_text)


# speedup vs baseline: 1.3510x; 1.0591x over previous
import jax
import jax.numpy as jnp
from jax import lax
from jax.experimental import pallas as pl
from jax.experimental.pallas import tpu as pltpu
from jax.experimental.pallas import tpu_sc as plsc

_D = 32
_W = 128
_K = 8
_L = 16


def _build(batch, seq):
    n = batch * seq
    num_rows = n // _W
    mesh = plsc.VectorSubcoreMesh(core_axis_name="c", subcore_axis_name="s")

    @jax.jit
    def run(table, input_text):
        idx2d = input_text.T.reshape(num_rows, _W)

        @pl.kernel(
            out_type=jax.ShapeDtypeStruct((n, _D), jnp.float32),
            mesh=mesh,
            scratch_types=[pltpu.SemaphoreType.DMA],
            compiler_params=pltpu.CompilerParams(use_tc_tiling_on_sc=False),
        )
        def gather_k(table_hbm, i_hbm, y_hbm, sem):
            def body(i_vmem, y_vmem):
                copies = [
                    pltpu.async_copy(
                        table_hbm.at[i_vmem.at[j]],
                        y_vmem.at[pl.ds(j * _W, _W)],
                        sem,
                    )
                    for j in range(_K)
                ]
                for c in copies:
                    c.wait()

            pltpu.emit_pipeline(
                body,
                grid=(num_rows // _K,),
                in_specs=[pl.BlockSpec((_K, _W), lambda i: (i, 0))],
                out_specs=[pl.BlockSpec((_K * _W, _D), lambda i: (i, 0))],
                core_axis_name=("c", "s"),
                dimension_semantics=(pltpu.PARALLEL,),
            )(i_hbm, y_hbm)

        y = gather_k(table, idx2d).reshape(n * _D)

        @pl.kernel(
            out_type=jax.ShapeDtypeStruct((seq, _D, batch), jnp.float32),
            mesh=mesh,
            compiler_params=pltpu.CompilerParams(
                use_tc_tiling_on_sc=True, needs_layout_passes=False
            ),
        )
        def transpose_k(y_hbm, x_hbm):
            def body(y_vmem, x_vmem):
                iota = lax.iota(jnp.int32, _L)
                zeros = iota * 0
                perms = [(iota + k) & (_L - 1) for k in range(_L)]
                pre_g = [iota * _D + p for p in perms]
                for j2 in range(_W // _L):
                    j_idx = iota + _L * j2
                    for h in range(_D // _L):
                        for k in range(_L):
                            g = pre_g[k] + (_L * _D * j2 + _L * h)
                            vals = plsc.load_gather(y_vmem, [g])
                            plsc.store_scatter(
                                x_vmem,
                                [zeros, perms[k] + _L * h, j_idx],
                                vals,
                            )

            pltpu.emit_pipeline(
                body,
                grid=(num_rows,),
                in_specs=[pl.BlockSpec((_W * _D,), lambda i: (i,))],
                out_specs=[
                    pl.BlockSpec(
                        (1, _D, _W),
                        lambda i: (i // (batch // _W), 0, i % (batch // _W)),
                    )
                ],
                core_axis_name=("c", "s"),
                dimension_semantics=(pltpu.PARALLEL,),
            )(y_hbm, x_hbm)

        x = transpose_k(y)
        return jnp.transpose(x, (2, 0, 1))

    return run


def kernel(input_text, embedding_table):
    batch, seq = input_text.shape
    run = _build(batch, seq)
    return run(embedding_table, input_text)


# speedup vs baseline: 1.6915x; 1.2520x over previous
import jax
import jax.numpy as jnp
from jax import lax
from jax.experimental import pallas as pl
from jax.experimental.pallas import tpu as pltpu
from jax.experimental.pallas import tpu_sc as plsc

_D = 32
_W = 128
_K = 8
_L = 16


def _build(batch, seq):
    n = batch * seq
    num_rows = n // _W
    mesh = plsc.VectorSubcoreMesh(core_axis_name="c", subcore_axis_name="s")

    @jax.jit
    def run(table, input_text):
        idx2d = input_text.T.reshape(num_rows, _W)

        @pl.kernel(
            out_type=jax.ShapeDtypeStruct((n, _D), jnp.float32),
            mesh=mesh,
            scratch_types=[pltpu.SemaphoreType.DMA],
            compiler_params=pltpu.CompilerParams(use_tc_tiling_on_sc=False),
        )
        def gather_k(table_hbm, i_hbm, y_hbm, sem):
            def body(i_vmem, y_vmem):
                copies = [
                    pltpu.async_copy(
                        table_hbm.at[i_vmem.at[j]],
                        y_vmem.at[pl.ds(j * _W, _W)],
                        sem,
                    )
                    for j in range(_K)
                ]
                for c in copies:
                    c.wait()

            pltpu.emit_pipeline(
                body,
                grid=(num_rows // _K,),
                in_specs=[pl.BlockSpec((_K, _W), lambda i: (i, 0))],
                out_specs=[pl.BlockSpec((_K * _W, _D), lambda i: (i, 0))],
                core_axis_name=("c", "s"),
                dimension_semantics=(pltpu.PARALLEL,),
            )(i_hbm, y_hbm)

        y = gather_k(table, idx2d).reshape(n * _D)

        @pl.kernel(
            out_type=jax.ShapeDtypeStruct((seq, _D, batch), jnp.float32),
            mesh=mesh,
            compiler_params=pltpu.CompilerParams(
                use_tc_tiling_on_sc=True, needs_layout_passes=False
            ),
        )
        def transpose_k(y_hbm, x_hbm):
            def body(y_vmem, x_vmem):
                base = lax.iota(jnp.int32, _L) * _D

                @plsc.parallel_loop(0, _D, unroll=2)
                def _(d):
                    row = base + d
                    for j in range(_W // _L):
                        vals = plsc.load_gather(y_vmem, [row + j * (_L * _D)])
                        x_vmem[0, d, pl.ds(j * _L, _L)] = vals

            pltpu.emit_pipeline(
                body,
                grid=(num_rows,),
                in_specs=[pl.BlockSpec((_W * _D,), lambda i: (i,))],
                out_specs=[
                    pl.BlockSpec(
                        (1, _D, _W),
                        lambda i: (i // (batch // _W), 0, i % (batch // _W)),
                    )
                ],
                core_axis_name=("c", "s"),
                dimension_semantics=(pltpu.PARALLEL,),
            )(y_hbm, x_hbm)

        x = transpose_k(y)
        return jnp.transpose(x, (2, 0, 1))

    return run


def kernel(input_text, embedding_table):
    batch, seq = input_text.shape
    run = _build(batch, seq)
    return run(embedding_table, input_text)


# speedup vs baseline: 1.7002x; 1.0051x over previous
import jax
import jax.numpy as jnp
from jax import lax
from jax.experimental import pallas as pl
from jax.experimental.pallas import tpu as pltpu
from jax.experimental.pallas import tpu_sc as plsc

_D = 32
_W = 128
_K = 8
_L = 16


def _build(batch, seq):
    n = batch * seq
    num_rows = n // _W
    mesh = plsc.VectorSubcoreMesh(core_axis_name="c", subcore_axis_name="s")

    @jax.jit
    def run(table, input_text):
        idx2d = input_text.T.reshape(num_rows, _W)

        @pl.kernel(
            out_type=jax.ShapeDtypeStruct((n, _D), jnp.float32),
            mesh=mesh,
            scratch_types=[pltpu.SemaphoreType.DMA],
            compiler_params=pltpu.CompilerParams(use_tc_tiling_on_sc=False),
        )
        def gather_k(table_hbm, i_hbm, y_hbm, sem):
            def body(i_vmem, y_vmem):
                copies = [
                    pltpu.async_copy(
                        table_hbm.at[i_vmem.at[j]],
                        y_vmem.at[pl.ds(j * _W, _W)],
                        sem,
                    )
                    for j in range(_K)
                ]
                for c in copies:
                    c.wait()

            pltpu.emit_pipeline(
                body,
                grid=(num_rows // _K,),
                in_specs=[pl.BlockSpec((_K, _W), lambda i: (i, 0))],
                out_specs=[pl.BlockSpec((_K * _W, _D), lambda i: (i, 0))],
                core_axis_name=("c", "s"),
                dimension_semantics=(pltpu.PARALLEL,),
            )(i_hbm, y_hbm)

        y = gather_k(table, idx2d).reshape(n * _D)

        @pl.kernel(
            out_type=jax.ShapeDtypeStruct((seq, _D, batch), jnp.float32),
            mesh=mesh,
            compiler_params=pltpu.CompilerParams(
                use_tc_tiling_on_sc=True, needs_layout_passes=False
            ),
        )
        def transpose_k(y_hbm, x_hbm):
            def body(y_vmem, x_vmem):
                base = lax.iota(jnp.int32, _L) * _D

                @plsc.parallel_loop(0, _D, unroll=4)
                def _(d):
                    row = base + d
                    for j in range(_W // _L):
                        vals = plsc.load_gather(y_vmem, [row + j * (_L * _D)])
                        x_vmem[0, d, pl.ds(j * _L, _L)] = vals

            pltpu.emit_pipeline(
                body,
                grid=(num_rows,),
                in_specs=[pl.BlockSpec((_W * _D,), lambda i: (i,))],
                out_specs=[
                    pl.BlockSpec(
                        (1, _D, _W),
                        lambda i: (i // (batch // _W), 0, i % (batch // _W)),
                    )
                ],
                core_axis_name=("c", "s"),
                dimension_semantics=(pltpu.PARALLEL,),
            )(y_hbm, x_hbm)

        x = transpose_k(y)
        return jnp.transpose(x, (2, 0, 1))

    return run


def kernel(input_text, embedding_table):
    batch, seq = input_text.shape
    run = _build(batch, seq)
    return run(embedding_table, input_text)


# speedup vs baseline: 3.2394x; 1.9053x over previous
import jax
import jax.numpy as jnp
from jax import lax
from jax.experimental import pallas as pl
from jax.experimental.pallas import tpu as pltpu
from jax.experimental.pallas import tpu_sc as plsc

_D = 32
_W = 128
_K = 8
_L = 16


def _build(batch, seq):
    n = batch * seq
    num_rows = n // _W
    mesh = plsc.VectorSubcoreMesh(core_axis_name="c", subcore_axis_name="s")

    @jax.jit
    def run(table, input_text):
        idx2d = input_text.T.reshape(num_rows, _W)

        @pl.kernel(
            out_type=jax.ShapeDtypeStruct((n, _D), jnp.float32),
            mesh=mesh,
            scratch_types=[pltpu.SemaphoreType.DMA],
            compiler_params=pltpu.CompilerParams(use_tc_tiling_on_sc=False),
        )
        def gather_k(table_hbm, i_hbm, y_hbm, sem):
            def body(i_vmem, y_vmem):
                copies = [
                    pltpu.async_copy(
                        table_hbm.at[i_vmem.at[j]],
                        y_vmem.at[pl.ds(j * _W, _W)],
                        sem,
                    )
                    for j in range(_K)
                ]
                for c in copies:
                    c.wait()

            pltpu.emit_pipeline(
                body,
                grid=(num_rows // _K,),
                in_specs=[pl.BlockSpec((_K, _W), lambda i: (i, 0))],
                out_specs=[pl.BlockSpec((_K * _W, _D), lambda i: (i, 0))],
                core_axis_name=("c", "s"),
                dimension_semantics=(pltpu.PARALLEL,),
            )(i_hbm, y_hbm)

        y = gather_k(table, idx2d).reshape(n * _D)

        @pl.kernel(
            out_type=jax.ShapeDtypeStruct((seq, _D, batch), jnp.float32),
            mesh=mesh,
            compiler_params=pltpu.CompilerParams(
                use_tc_tiling_on_sc=True, needs_layout_passes=False
            ),
        )
        def transpose_k(y_hbm, x_hbm):
            def body(y_vmem, x_vmem):
                iota = lax.iota(jnp.int32, _L)
                zeros = iota * 0
                jvecs = [iota + _L * j2 for j2 in range(_W // _L)]

                @plsc.parallel_loop(0, _L, unroll=2)
                def _(k):
                    perm = (iota + k) & (_L - 1)
                    pre_g = iota * _D + perm
                    for h in range(_D // _L):
                        d_idx = perm + _L * h
                        for j2 in range(_W // _L):
                            g = pre_g + (_L * _D * j2 + _L * h)
                            vals = plsc.load_gather(y_vmem, [g])
                            plsc.store_scatter(
                                x_vmem, [zeros, d_idx, jvecs[j2]], vals
                            )

            pltpu.emit_pipeline(
                body,
                grid=(num_rows,),
                in_specs=[pl.BlockSpec((_W * _D,), lambda i: (i,))],
                out_specs=[
                    pl.BlockSpec(
                        (1, _D, _W),
                        lambda i: (i // (batch // _W), 0, i % (batch // _W)),
                    )
                ],
                core_axis_name=("c", "s"),
                dimension_semantics=(pltpu.PARALLEL,),
            )(y_hbm, x_hbm)

        x = transpose_k(y)
        return jnp.transpose(x, (2, 0, 1))

    return run


def kernel(input_text, embedding_table):
    batch, seq = input_text.shape
    run = _build(batch, seq)
    return run(embedding_table, input_text)


# speedup vs baseline: 3.2439x; 1.0014x over previous
import jax
import jax.numpy as jnp
from jax import lax
from jax.experimental import pallas as pl
from jax.experimental.pallas import tpu as pltpu
from jax.experimental.pallas import tpu_sc as plsc

_D = 32
_W = 128
_K = 8
_L = 16


def _build(batch, seq):
    n = batch * seq
    num_rows = n // _W
    mesh = plsc.VectorSubcoreMesh(core_axis_name="c", subcore_axis_name="s")

    @jax.jit
    def run(table, input_text):
        idx2d = input_text.T.reshape(num_rows, _W)

        @pl.kernel(
            out_type=jax.ShapeDtypeStruct((n, _D), jnp.float32),
            mesh=mesh,
            scratch_types=[pltpu.SemaphoreType.DMA],
            compiler_params=pltpu.CompilerParams(use_tc_tiling_on_sc=False),
        )
        def gather_k(table_hbm, i_hbm, y_hbm, sem):
            def body(i_vmem, y_vmem):
                copies = [
                    pltpu.async_copy(
                        table_hbm.at[i_vmem.at[j]],
                        y_vmem.at[pl.ds(j * _W, _W)],
                        sem,
                    )
                    for j in range(_K)
                ]
                for c in copies:
                    c.wait()

            pltpu.emit_pipeline(
                body,
                grid=(num_rows // _K,),
                in_specs=[pl.BlockSpec((_K, _W), lambda i: (i, 0))],
                out_specs=[pl.BlockSpec((_K * _W, _D), lambda i: (i, 0))],
                core_axis_name=("c", "s"),
                dimension_semantics=(pltpu.PARALLEL,),
            )(i_hbm, y_hbm)

        y = gather_k(table, idx2d).reshape(n * _D)

        @pl.kernel(
            out_type=jax.ShapeDtypeStruct((seq, _D, batch), jnp.float32),
            mesh=mesh,
            compiler_params=pltpu.CompilerParams(
                use_tc_tiling_on_sc=True, needs_layout_passes=False
            ),
        )
        def transpose_k(y_hbm, x_hbm):
            def body(y_vmem, x_vmem):
                iota = lax.iota(jnp.int32, _L)
                zeros = iota * 0
                jvecs = [iota + _L * j2 for j2 in range(_W // _L)]

                @plsc.parallel_loop(0, _L, unroll=4)
                def _(k):
                    perm = (iota + k) & (_L - 1)
                    pre_g = iota * _D + perm
                    for h in range(_D // _L):
                        d_idx = perm + _L * h
                        for j2 in range(_W // _L):
                            g = pre_g + (_L * _D * j2 + _L * h)
                            vals = plsc.load_gather(y_vmem, [g])
                            plsc.store_scatter(
                                x_vmem, [zeros, d_idx, jvecs[j2]], vals
                            )

            pltpu.emit_pipeline(
                body,
                grid=(num_rows,),
                in_specs=[pl.BlockSpec((_W * _D,), lambda i: (i,))],
                out_specs=[
                    pl.BlockSpec(
                        (1, _D, _W),
                        lambda i: (i // (batch // _W), 0, i % (batch // _W)),
                    )
                ],
                core_axis_name=("c", "s"),
                dimension_semantics=(pltpu.PARALLEL,),
            )(y_hbm, x_hbm)

        x = transpose_k(y)
        return jnp.transpose(x, (2, 0, 1))

    return run


def kernel(input_text, embedding_table):
    batch, seq = input_text.shape
    run = _build(batch, seq)
    return run(embedding_table, input_text)


# speedup vs baseline: 3.2499x; 1.0019x over previous
import jax
import jax.numpy as jnp
from jax import lax
from jax.experimental import pallas as pl
from jax.experimental.pallas import tpu as pltpu
from jax.experimental.pallas import tpu_sc as plsc

_D = 32
_W = 128
_K = 10
_L = 16


def _build(batch, seq):
    n = batch * seq
    num_rows = n // _W
    mesh = plsc.VectorSubcoreMesh(core_axis_name="c", subcore_axis_name="s")

    @jax.jit
    def run(table, input_text):
        idx2d = input_text.T.reshape(num_rows, _W)

        @pl.kernel(
            out_type=jax.ShapeDtypeStruct((n, _D), jnp.float32),
            mesh=mesh,
            scratch_types=[pltpu.SemaphoreType.DMA],
            compiler_params=pltpu.CompilerParams(use_tc_tiling_on_sc=False),
        )
        def gather_k(table_hbm, i_hbm, y_hbm, sem):
            def body(i_vmem, y_vmem):
                copies = [
                    pltpu.async_copy(
                        table_hbm.at[i_vmem.at[j]],
                        y_vmem.at[pl.ds(j * _W, _W)],
                        sem,
                    )
                    for j in range(_K)
                ]
                for c in copies:
                    c.wait()

            pltpu.emit_pipeline(
                body,
                grid=(num_rows // _K,),
                in_specs=[pl.BlockSpec((_K, _W), lambda i: (i, 0))],
                out_specs=[pl.BlockSpec((_K * _W, _D), lambda i: (i, 0))],
                core_axis_name=("c", "s"),
                dimension_semantics=(pltpu.PARALLEL,),
            )(i_hbm, y_hbm)

        y = gather_k(table, idx2d).reshape(n * _D)

        @pl.kernel(
            out_type=jax.ShapeDtypeStruct((seq, _D, batch), jnp.float32),
            mesh=mesh,
            compiler_params=pltpu.CompilerParams(
                use_tc_tiling_on_sc=True, needs_layout_passes=False
            ),
        )
        def transpose_k(y_hbm, x_hbm):
            def body(y_vmem, x_vmem):
                iota = lax.iota(jnp.int32, _L)
                zeros = iota * 0
                jvecs = [iota + _L * j2 for j2 in range(_W // _L)]

                @plsc.parallel_loop(0, _L, unroll=4)
                def _(k):
                    perm = (iota + k) & (_L - 1)
                    pre_g = iota * _D + perm
                    for h in range(_D // _L):
                        d_idx = perm + _L * h
                        for j2 in range(_W // _L):
                            g = pre_g + (_L * _D * j2 + _L * h)
                            vals = plsc.load_gather(y_vmem, [g])
                            plsc.store_scatter(
                                x_vmem, [zeros, d_idx, jvecs[j2]], vals
                            )

            pltpu.emit_pipeline(
                body,
                grid=(num_rows,),
                in_specs=[pl.BlockSpec((_W * _D,), lambda i: (i,))],
                out_specs=[
                    pl.BlockSpec(
                        (1, _D, _W),
                        lambda i: (i // (batch // _W), 0, i % (batch // _W)),
                    )
                ],
                core_axis_name=("c", "s"),
                dimension_semantics=(pltpu.PARALLEL,),
            )(y_hbm, x_hbm)

        x = transpose_k(y)
        return jnp.transpose(x, (2, 0, 1))

    return run


def kernel(input_text, embedding_table):
    batch, seq = input_text.shape
    run = _build(batch, seq)
    return run(embedding_table, input_text)
